# Initial kernel scaffold; baseline (speedup 1.0000x reference)
import jax
import jax.numpy as jnp
from jax.experimental import pallas as pl


def kernel(x, Wg, bg, W1, b1, W2, b2):
    raise NotImplementedError("write your pallas kernel here")


# speedup vs baseline: 2.1270x; 2.1270x over previous
import functools

import jax
import jax.numpy as jnp
from jax.experimental import pallas as pl
from jax.experimental.pallas import tpu as pltpu

_TOPK = 2
_T = 256
_FT = 512
_INTERPRET = False


def _router_body(x_ref, wg_ref, bg_ref, eidx_ref, wgt_ref):
    x = x_ref[...]
    logits = jnp.dot(x, wg_ref[...], preferred_element_type=jnp.float32)
    logits = logits + bg_ref[...]
    m = jnp.max(logits, axis=1, keepdims=True)
    ex = jnp.exp(logits - m)
    probs = ex / jnp.sum(ex, axis=1, keepdims=True)
    lane = jax.lax.broadcasted_iota(jnp.int32, probs.shape, 1)
    big = jnp.int32(10**6)
    m0 = jnp.max(probs, axis=1, keepdims=True)
    i0 = jnp.min(jnp.where(probs == m0, lane, big), axis=1, keepdims=True)
    probs1 = jnp.where(lane == i0, -1.0, probs)
    m1 = jnp.max(probs1, axis=1, keepdims=True)
    i1 = jnp.min(jnp.where(probs1 == m1, lane, big), axis=1, keepdims=True)
    eidx_ref[...] = jnp.where(lane == 0, i0, jnp.where(lane == 1, i1, 0))
    wgt_ref[...] = jnp.where(lane == 0, m0, jnp.where(lane == 1, m1, 0.0))


def _router(x_flat, Wg, bg):
    n, d = x_flat.shape
    e = Wg.shape[1]
    rt = 512
    wg_pad = jnp.zeros((d, 128), jnp.float32).at[:, :e].set(Wg)
    bg_pad = jnp.full((1, 128), -1e30, jnp.float32).at[0, :e].set(bg)
    eidx, wgt = pl.pallas_call(
        _router_body,
        grid=(n // rt,),
        in_specs=[
            pl.BlockSpec((rt, d), lambda i: (i, 0)),
            pl.BlockSpec((d, 128), lambda i: (0, 0)),
            pl.BlockSpec((1, 128), lambda i: (0, 0)),
        ],
        out_specs=[
            pl.BlockSpec((rt, 128), lambda i: (i, 0)),
            pl.BlockSpec((rt, 128), lambda i: (i, 0)),
        ],
        out_shape=[
            jax.ShapeDtypeStruct((n, 128), jnp.int32),
            jax.ShapeDtypeStruct((n, 128), jnp.float32),
        ],
        interpret=_INTERPRET,
    )(x_flat, wg_pad, bg_pad)
    return eidx[:, :_TOPK], wgt[:, :_TOPK]


def _erf(z):
    s = jnp.sign(z)
    a = jnp.abs(z)
    t = 1.0 / (1.0 + 0.3275911 * a)
    poly = t * (0.254829592 + t * (-0.284496736 + t * (1.421413741
           + t * (-1.453152027 + t * 1.061405429))))
    return s * (1.0 - poly * jnp.exp(-a * a))


def _gelu(h):
    return 0.5 * h * (1.0 + _erf(h * 0.7071067811865476))


def _ffn_body(nf, be_ref, xs_ref, w1_ref, b1_ref, w2_ref, b2_ref, gw_ref,
              out_ref):
    f = pl.program_id(1)
    h = jnp.dot(xs_ref[...], w1_ref[0], preferred_element_type=jnp.float32)
    h = _gelu(h + b1_ref[0])
    acc = jnp.dot(h, w2_ref[0], preferred_element_type=jnp.float32)

    @pl.when(f == 0)
    def _():
        out_ref[...] = acc

    @pl.when(f > 0)
    def _():
        out_ref[...] = out_ref[...] + acc

    @pl.when(f == nf - 1)
    def _():
        out_ref[...] = (out_ref[...] + b2_ref[0]) * gw_ref[...]


def _ffn(xs, W1, b1, W2, b2, gw, be):
    ns, d = xs.shape
    e, _, f_dim = W1.shape
    nb = ns // _T
    nf = f_dim // _FT
    grid_spec = pltpu.PrefetchScalarGridSpec(
        num_scalar_prefetch=1,
        grid=(nb, nf),
        in_specs=[
            pl.BlockSpec((_T, d), lambda b, f, be: (b, 0)),
            pl.BlockSpec((1, d, _FT), lambda b, f, be: (be[b], 0, f)),
            pl.BlockSpec((1, 1, _FT), lambda b, f, be: (be[b], 0, f)),
            pl.BlockSpec((1, _FT, d), lambda b, f, be: (be[b], f, 0)),
            pl.BlockSpec((1, 1, d), lambda b, f, be: (be[b], 0, 0)),
            pl.BlockSpec((_T, 1), lambda b, f, be: (b, 0)),
        ],
        out_specs=pl.BlockSpec((_T, d), lambda b, f, be: (b, 0)),
    )
    return pl.pallas_call(
        functools.partial(_ffn_body, nf),
        grid_spec=grid_spec,
        out_shape=jax.ShapeDtypeStruct((ns, d), jnp.float32),
        compiler_params=pltpu.CompilerParams(
            dimension_semantics=("arbitrary", "arbitrary")),
        interpret=_INTERPRET,
    )(be, xs, W1, b1.reshape(e, 1, f_dim), W2, b2.reshape(e, 1, d), gw)


def kernel(x, Wg, bg, W1, b1, W2, b2):
    b, s, d = x.shape
    e = Wg.shape[1]
    n = b * s
    nk = n * _TOPK
    ns = nk + e * _T
    x_flat = x.reshape(n, d)

    eidx, wgt = _router(x_flat, Wg, bg)

    ef = jnp.concatenate([eidx[:, 0], eidx[:, 1]])
    wf = jnp.concatenate([wgt[:, 0], wgt[:, 1]])
    oh = (ef[:, None] == jnp.arange(e, dtype=jnp.int32)[None, :]).astype(
        jnp.int32)
    csum = jnp.cumsum(oh, axis=0)
    rank = jnp.sum((csum - oh) * oh, axis=1)
    counts = csum[-1]
    padded = ((counts + _T - 1) // _T) * _T
    ends = jnp.cumsum(padded)
    seg = ends - padded
    slot = seg[ef] + rank
    token = jnp.concatenate(
        [jnp.arange(n, dtype=jnp.int32)] * 2)
    token_src = jnp.zeros((ns,), jnp.int32).at[slot].set(token)
    gw = jnp.zeros((ns, 1), jnp.float32).at[slot, 0].set(wf)
    blk_start = jnp.arange(ns // _T, dtype=jnp.int32) * _T
    be = jnp.minimum(
        jnp.sum((blk_start[:, None] >= ends[None, :]).astype(jnp.int32),
                axis=1), e - 1).astype(jnp.int32)

    xs = x_flat[token_src]
    ys = _ffn(xs, W1, b1, W2, b2, gw, be)
    y = ys[slot[:n]] + ys[slot[n:]]
    return y.reshape(b, s, d)
